# baseline (speedup 1.0000x reference)
import functools

import jax
import jax.numpy as jnp
from jax import lax
from jax.experimental import pallas as pl
from jax.experimental.pallas import tpu as pltpu
from jax.experimental.pallas import tpu_sc as plsc

_NC = 2
_NS = 16
_NW = _NC * _NS


@functools.lru_cache(maxsize=None)
def _make_lookup(V, D, BB, T):
    DG = D // 8
    BPW = BB // _NW
    NBT = BPW // 128
    NU = T * NBT
    mesh = plsc.VectorSubcoreMesh(
        core_axis_name="c", subcore_axis_name="s", num_cores=_NC, num_subcores=_NS
    )

    @functools.partial(
        pl.kernel,
        out_type=jax.ShapeDtypeStruct((BB * T * D,), jnp.float32),
        mesh=mesh,
        scratch_types=[
            pltpu.VMEM((T, BPW), jnp.int32),
            [pltpu.VMEM((128,), jnp.int32) for _ in range(2)],
            [pltpu.VMEM((128, 4 * D), jnp.float32) for _ in range(2)],
            [pltpu.VMEM((DG * 1024,), jnp.float32) for _ in range(2)],
            [pltpu.SemaphoreType.DMA for _ in range(2)],
            [pltpu.SemaphoreType.DMA for _ in range(2)],
        ],
        compiler_params=pltpu.CompilerParams(
            use_tc_tiling_on_sc=False, needs_layout_passes=False
        ),
    )
    def lookup_kernel(
        idx_hbm, table_hbm, out_hbm, idx_t, idx4, rows, outs, gsems, wsems
    ):
        wid = lax.axis_index("s") * _NC + lax.axis_index("c")
        iota = lax.iota(jnp.int32, 16)

        pltpu.sync_copy(idx_hbm.at[:, pl.ds(wid * BPW, BPW)], idx_t)

        def unit(u):
            return u // NBT, u % NBT

        def woff(t, bt, dg):
            return ((t * DG + dg) * (BB // 128) + NBT * wid + bt) * 1024

        def start_gather(u, p):
            t, bt = unit(u)
            for lc in range(8):
                v = idx_t[t, pl.ds(bt * 128 + lc * 16, 16)]
                idx4[p][pl.ds(lc * 16, 16)] = lax.shift_right_logical(v, 2)
            pltpu.async_copy(table_hbm.at[idx4[p]], rows[p], gsems[p])

        def process(u, p):
            t, bt = unit(u)
            outbuf, wsem = outs[p], wsems[p]

            @pl.when(u >= 2)
            def _():
                for dg in range(DG):
                    pltpu.make_async_copy(
                        outbuf.at[pl.ds(dg * 1024, 1024)],
                        out_hbm.at[pl.ds(woff(t, bt, dg), 1024)],
                        wsem,
                    ).wait()

            pltpu.make_async_copy(
                table_hbm.at[idx4[p]], rows[p], gsems[p]
            ).wait()

            rids = [lc * 16 + iota for lc in range(8)]
            quads = [
                (idx_t[t, pl.ds(bt * 128 + lc * 16, 16)] & 3) * D
                for lc in range(8)
            ]
            for dg in range(DG):
                for s in range(8):
                    dcol = dg * 8 + s
                    vs = [
                        plsc.load_gather(rows[p], [rids[lc], quads[lc] + dcol])
                        for lc in range(8)
                    ]
                    for lc in range(8):
                        outbuf[pl.ds(dg * 1024 + s * 128 + lc * 16, 16)] = vs[lc]

            for dg in range(DG):
                pltpu.async_copy(
                    outbuf.at[pl.ds(dg * 1024, 1024)],
                    out_hbm.at[pl.ds(woff(t, bt, dg), 1024)],
                    wsem,
                )

            @pl.when(u + 2 < NU)
            def _():
                start_gather(u + 2, p)

        start_gather(0, 0)
        start_gather(1, 1)

        def step(j, carry):
            process(2 * j, 0)
            process(2 * j + 1, 1)
            return carry

        lax.fori_loop(0, NU // 2, step, 0)

        for p, u in ((0, NU - 2), (1, NU - 1)):
            t, bt = unit(u)
            for dg in range(DG):
                pltpu.make_async_copy(
                    outs[p].at[pl.ds(dg * 1024, 1024)],
                    out_hbm.at[pl.ds(woff(t, bt, dg), 1024)],
                    wsems[p],
                ).wait()

    return lookup_kernel


def kernel(inputs, weight):
    BB, T = inputs.shape
    V, D = weight.shape
    idx_tm = inputs.T.astype(jnp.int32)
    table4 = weight.reshape(V // 4, 4 * D)
    out1d = _make_lookup(V, D, BB, T)(idx_tm, table4)
    s5 = out1d.reshape(T, D // 8, BB // 128, 8, 128)
    return s5.transpose(2, 4, 0, 1, 3).reshape(BB, T, D)
